# Initial kernel scaffold; baseline (speedup 1.0000x reference)
import jax
import jax.numpy as jnp
from jax.experimental import pallas as pl


def kernel(x, edge_index, batch, W1, b1, W2, b2, lin_W, lin_b):
    raise NotImplementedError("write your pallas kernel here")


# speedup vs baseline: 16.3955x; 16.3955x over previous
import functools

import jax
import jax.numpy as jnp
from jax import lax
from jax.experimental import pallas as pl
from jax.experimental.pallas import tpu as pltpu
from jax.experimental.pallas import tpu_sc as plsc

N = 10000
E = 320000
D = 128
G = 16
NC, NS = 2, 16
NW = NC * NS
E_W = E // NW
CH = 128
NFULL = E_W // CH
TAIL = E_W - NFULL * CH

_mesh = plsc.VectorSubcoreMesh(core_axis_name="c", subcore_axis_name="s")


@functools.partial(
    pl.kernel,
    mesh=_mesh,
    out_type=jax.ShapeDtypeStruct((NC * N,), jnp.float32),
    scratch_types=[
        pltpu.VMEM((CH,), jnp.int32),
        pltpu.VMEM((TAIL,), jnp.int32),
        pltpu.VMEM((CH,), jnp.float32),
        pltpu.VMEM((TAIL,), jnp.float32),
        pltpu.VMEM((N,), jnp.float32),
        pltpu.VMEM_SHARED((N,), jnp.float32),
    ],
)
def _deg_kernel(dst_hbm, zero_hbm, out_hbm, idx_v, idxt_v, ones_v, onest_v,
                bounce_v, acc_s):
    c = lax.axis_index("c")
    s = lax.axis_index("s")
    w = c * NS + s
    for k in range(CH // 16):
        ones_v[pl.ds(k * 16, 16)] = jnp.ones((16,), jnp.float32)
    onest_v[pl.ds(0, 16)] = jnp.ones((16,), jnp.float32)

    @pl.when(s == 0)
    def _():
        pltpu.sync_copy(zero_hbm, bounce_v)
        pltpu.sync_copy(bounce_v, acc_s)

    plsc.subcore_barrier()
    base = w * E_W

    def body(j, carry):
        pltpu.sync_copy(dst_hbm.at[pl.ds(base + j * CH, CH)], idx_v)
        pltpu.sync_copy(ones_v, acc_s.at[idx_v], add=True)
        return carry

    lax.fori_loop(0, NFULL, body, 0)
    pltpu.sync_copy(dst_hbm.at[pl.ds(base + NFULL * CH, TAIL)], idxt_v)
    pltpu.sync_copy(onest_v, acc_s.at[idxt_v], add=True)
    plsc.subcore_barrier()

    @pl.when(s == 0)
    def _():
        pltpu.sync_copy(acc_s, bounce_v)
        pltpu.sync_copy(bounce_v,
                        out_hbm.at[pl.ds(pl.multiple_of(c * N, 8), N)])


@functools.partial(
    pl.kernel,
    mesh=_mesh,
    out_type=jax.ShapeDtypeStruct((NC, N, D), jnp.float32),
    scratch_types=[
        pltpu.VMEM((CH,), jnp.int32),
        pltpu.VMEM((CH,), jnp.int32),
        pltpu.VMEM((TAIL,), jnp.int32),
        pltpu.VMEM((TAIL,), jnp.int32),
        pltpu.VMEM((CH, D), jnp.float32),
        pltpu.VMEM((TAIL, D), jnp.float32),
        pltpu.VMEM_SHARED((N, D), jnp.float32),
        pltpu.SemaphoreType.DMA,
    ],
)
def _agg_kernel(xs_hbm, src_hbm, dst_hbm, zero_hbm, out_hbm, si_v, di_v,
                sit_v, dit_v, rows_v, rowst_v, acc_s, sem):
    c = lax.axis_index("c")
    s = lax.axis_index("s")
    w = c * NS + s

    @pl.when(s == 0)
    def _():
        pltpu.sync_copy(zero_hbm, acc_s)

    plsc.subcore_barrier()

    base = w * E_W

    def body(j, carry):
        off = base + j * CH
        pltpu.sync_copy(src_hbm.at[pl.ds(off, CH)], si_v)
        pltpu.async_copy(xs_hbm.at[si_v], rows_v, sem).wait()
        pltpu.sync_copy(dst_hbm.at[pl.ds(off, CH)], di_v)
        pltpu.sync_copy(rows_v, acc_s.at[di_v], add=True)
        return carry

    lax.fori_loop(0, NFULL, body, 0)
    off = base + NFULL * CH
    pltpu.sync_copy(src_hbm.at[pl.ds(off, TAIL)], sit_v)
    pltpu.async_copy(xs_hbm.at[sit_v], rowst_v, sem).wait()
    pltpu.sync_copy(dst_hbm.at[pl.ds(off, TAIL)], dit_v)
    pltpu.sync_copy(rowst_v, acc_s.at[dit_v], add=True)
    plsc.subcore_barrier()

    @pl.when(s == 0)
    def _():
        pltpu.sync_copy(acc_s, out_hbm.at[c])


def _scale_body(deg_ref, x_ref, xs_ref):
    dinv = lax.rsqrt(deg_ref[:, 0:1] + deg_ref[:, 1:2] + 1.0)
    xs_ref[...] = x_ref[...] * dinv


def _layer_body(acc_ref, deg_ref, xin_ref, w_ref, b_ref, h_ref, xs_ref):
    dinv = lax.rsqrt(deg_ref[:, 0:1] + deg_ref[:, 1:2] + 1.0)
    agg = dinv * (acc_ref[0] + acc_ref[1]) + (dinv * dinv) * xin_ref[...]
    h = jnp.dot(agg, w_ref[...], preferred_element_type=jnp.float32)
    h = jnp.maximum(h + b_ref[...], 0.0)
    h_ref[...] = h
    xs_ref[...] = h * dinv


def _final_body(acc_ref, deg_ref, h1_ref, w_ref, b_ref, batch_ref, lw_ref,
                lb_ref, out_ref):
    dinv = lax.rsqrt(deg_ref[:, 0:1] + deg_ref[:, 1:2] + 1.0)
    agg = dinv * (acc_ref[0] + acc_ref[1]) + (dinv * dinv) * h1_ref[...]
    h2 = jnp.dot(agg, w_ref[...], preferred_element_type=jnp.float32)
    h2 = jnp.maximum(h2 + b_ref[...], 0.0)
    gid = lax.broadcasted_iota(jnp.int32, (G, N), 0)
    onehot = (jnp.broadcast_to(batch_ref[...], (G, N)) == gid)
    onehot = onehot.astype(jnp.float32)
    sums = jnp.dot(onehot, h2, preferred_element_type=jnp.float32)
    counts = jnp.sum(onehot, axis=1, keepdims=True)
    pooled = sums / jnp.maximum(counts, 1.0)
    out_ref[...] = (
        jnp.dot(pooled, lw_ref[...], preferred_element_type=jnp.float32)
        + lb_ref[...])


_scale_call = pl.pallas_call(
    _scale_body, out_shape=jax.ShapeDtypeStruct((N, D), jnp.float32))

_layer_call = pl.pallas_call(
    _layer_body,
    out_shape=(jax.ShapeDtypeStruct((N, D), jnp.float32),
               jax.ShapeDtypeStruct((N, D), jnp.float32)))

_final_call = pl.pallas_call(
    _final_body, out_shape=jax.ShapeDtypeStruct((G, 1), jnp.float32))


@jax.jit
def kernel(x, edge_index, batch, W1, b1, W2, b2, lin_W, lin_b):
    src = edge_index[0]
    dst = edge_index[1]
    zero1 = jnp.zeros((N,), jnp.float32)
    zero2 = jnp.zeros((N, D), jnp.float32)
    deg_t = _deg_kernel(dst, zero1).reshape(NC, N).T
    xs1 = _scale_call(deg_t, x)
    acc1 = _agg_kernel(xs1, src, dst, zero2)
    h1, xs2 = _layer_call(acc1, deg_t, x, W1, b1)
    acc2 = _agg_kernel(xs2, src, dst, zero2)
    return _final_call(acc2, deg_t, h1, W2, b2, batch.reshape(1, N),
                       lin_W, lin_b)
